# Initial kernel scaffold; baseline (speedup 1.0000x reference)
import jax
import jax.numpy as jnp
from jax.experimental import pallas as pl


def kernel(x, edge_index, edge_attr, params):
    raise NotImplementedError("write your pallas kernel here")


# speedup vs baseline: 1.6311x; 1.6311x over previous
import functools

import jax
import jax.numpy as jnp
from jax import lax
from jax.experimental import pallas as pl
from jax.experimental.pallas import tpu as pltpu
from jax.experimental.pallas import tpu_sc as plsc

N = 50000
E = 800000
NODE_DIM = 25
HID = 64
HALF = 32


_BN = 1000
_BE = 1000


def _proj_body(x_ref, w_ref, b_ref, lo_ref, hi_ref):
    h = jnp.dot(x_ref[...], w_ref[...], preferred_element_type=jnp.float32)
    h = jnp.maximum(h + b_ref[...], 0.0)
    lo_ref[...] = h[:, :HALF]
    hi_ref[...] = h[:, HALF:]


def _edge_mlp_body(ea_ref, *refs):
    a = ea_ref[...]
    for l in range(3):
        w1, b1, w2, b2 = refs[4 * l: 4 * l + 4]
        t = jnp.maximum(
            jnp.dot(a, w1[...], preferred_element_type=jnp.float32) + b1[...], 0.0)
        y = jnp.dot(t, w2[...], preferred_element_type=jnp.float32) + b2[...]
        refs[12 + 2 * l][...] = y[:, :HALF]
        refs[12 + 2 * l + 1][...] = y[:, HALF:]


def _node_body(hlo_ref, hhi_ref, alo_ref, ahi_ref, ws_ref, wn_ref, b_ref,
               lo_ref, hi_ref):
    h = jnp.concatenate([hlo_ref[...], hhi_ref[...]], axis=1)
    a = jnp.concatenate([alo_ref[...], ahi_ref[...]], axis=1)
    y = (jnp.dot(h, ws_ref[...], preferred_element_type=jnp.float32)
         + jnp.dot(a, wn_ref[...], preferred_element_type=jnp.float32)
         + b_ref[...])
    y = jnp.maximum(y, 0.0)
    lo_ref[...] = y[:, :HALF]
    hi_ref[...] = y[:, HALF:]


def _node_mean_body(hlo_ref, hhi_ref, alo_ref, ahi_ref, ws_ref, wn_ref, b_ref,
                    out_ref):
    i = pl.program_id(0)
    h = jnp.concatenate([hlo_ref[...], hhi_ref[...]], axis=1)
    a = jnp.concatenate([alo_ref[...], ahi_ref[...]], axis=1)
    y = (jnp.dot(h, ws_ref[...], preferred_element_type=jnp.float32)
         + jnp.dot(a, wn_ref[...], preferred_element_type=jnp.float32)
         + b_ref[...])
    y = jnp.maximum(y, 0.0)

    @pl.when(i == 0)
    def _():
        out_ref[...] = jnp.zeros_like(out_ref)

    out_ref[0:1, :] += jnp.sum(y, axis=0, keepdims=True)

    @pl.when(i == pl.num_programs(0) - 1)
    def _():
        out_ref[0:1, :] *= (1.0 / N)


def _full(shape):
    return pl.BlockSpec(shape, lambda i: (0,) * len(shape))


def _tc_proj(x, wp, bp):
    g = N // _BN
    return pl.pallas_call(
        _proj_body,
        grid=(g,),
        in_specs=[
            pl.BlockSpec((_BN, NODE_DIM), lambda i: (i, 0)),
            _full((NODE_DIM, HID)),
            _full((1, HID)),
        ],
        out_specs=[pl.BlockSpec((_BN, HALF), lambda i: (i, 0))] * 2,
        out_shape=[jax.ShapeDtypeStruct((N, HALF), jnp.float32)] * 2,
    )(x, wp, bp)


def _tc_edge_mlp(ea, wargs):
    g = E // _BE
    in_specs = [pl.BlockSpec((_BE, 17), lambda i: (i, 0))]
    for _ in range(3):
        in_specs += [_full((17, HID)), _full((1, HID)),
                     _full((HID, HID)), _full((1, HID))]
    return pl.pallas_call(
        _edge_mlp_body,
        grid=(g,),
        in_specs=in_specs,
        out_specs=[pl.BlockSpec((_BE, HALF), lambda i: (i, 0))] * 6,
        out_shape=[jax.ShapeDtypeStruct((E, HALF), jnp.float32)] * 6,
    )(ea, *wargs)


def _tc_node(hlo, hhi, alo, ahi, ws, wn, b):
    g = N // _BN
    return pl.pallas_call(
        _node_body,
        grid=(g,),
        in_specs=[pl.BlockSpec((_BN, HALF), lambda i: (i, 0))] * 4
        + [_full((HID, HID)), _full((HID, HID)), _full((1, HID))],
        out_specs=[pl.BlockSpec((_BN, HALF), lambda i: (i, 0))] * 2,
        out_shape=[jax.ShapeDtypeStruct((N, HALF), jnp.float32)] * 2,
    )(hlo, hhi, alo, ahi, ws, wn, b)


def _tc_node_mean(hlo, hhi, alo, ahi, ws, wn, b):
    g = N // _BN
    return pl.pallas_call(
        _node_mean_body,
        grid=(g,),
        in_specs=[pl.BlockSpec((_BN, HALF), lambda i: (i, 0))] * 4
        + [_full((HID, HID)), _full((HID, HID)), _full((1, HID))],
        out_specs=pl.BlockSpec((8, HID), lambda i: (0, 0)),
        out_shape=jax.ShapeDtypeStruct((8, HID), jnp.float32),
    )(hlo, hhi, alo, ahi, ws, wn, b)


_C = 128
_NCHUNKS = E // _C
_TILES = 16
_ITERS = -(-_NCHUNKS // _TILES)
_ZROWS = 40
_ZCHUNKS = N // _ZROWS
_ZITERS = -(-_ZCHUNKS // _TILES)


def _sc_body(hlo_hbm, hhi_hbm, wlo_hbm, whi_hbm, src_hbm, dst_hbm,
             olo_hbm, ohi_hbm,
             sidx, didx, hrows, wrows, zbuf, acc, sem):
    cid = lax.axis_index("c")
    sid = lax.axis_index("s")

    def _zb(r, _):
        for c in range(HALF // 16):
            zbuf[r, pl.ds(16 * c, 16)] = jnp.zeros((16,), jnp.float32)
        return _
    lax.fori_loop(0, _ZROWS, _zb, None)

    def _zero(j, _):
        ch = j * _TILES + sid

        @pl.when(ch < _ZCHUNKS)
        def _():
            pltpu.sync_copy(zbuf, acc.at[pl.ds(ch * _ZROWS, _ZROWS)])
        return _
    lax.fori_loop(0, _ZITERS, _zero, None)

    plsc.subcore_barrier()

    def _edges(j, _):
        g = j * _TILES + sid

        @pl.when(g < _NCHUNKS)
        def _():
            base = g * _C
            pltpu.sync_copy(src_hbm.at[pl.ds(base, _C)], sidx)
            pltpu.sync_copy(dst_hbm.at[pl.ds(base, _C)], didx)

            @pl.when(cid == 0)
            def _():
                pltpu.async_copy(hlo_hbm.at[sidx], hrows, sem).wait()
                pltpu.sync_copy(wlo_hbm.at[pl.ds(base, _C)], wrows)

            @pl.when(cid == 1)
            def _():
                pltpu.async_copy(hhi_hbm.at[sidx], hrows, sem).wait()
                pltpu.sync_copy(whi_hbm.at[pl.ds(base, _C)], wrows)

            def _mul(r, _):
                for c in range(HALF // 16):
                    sl = pl.ds(16 * c, 16)
                    wrows[r, sl] = wrows[r, sl] * hrows[r, sl]
                return _
            lax.fori_loop(0, _C, _mul, None)

            pltpu.sync_copy(wrows, acc.at[didx], add=True)
        return _
    lax.fori_loop(0, _ITERS, _edges, None)

    plsc.subcore_barrier()

    def _wb(j, _):
        ch = j * _TILES + sid

        @pl.when(ch < _ZCHUNKS)
        def _():
            rows = pl.ds(ch * _ZROWS, _ZROWS)

            @pl.when(cid == 0)
            def _():
                pltpu.sync_copy(acc.at[rows], olo_hbm.at[rows])

            @pl.when(cid == 1)
            def _():
                pltpu.sync_copy(acc.at[rows], ohi_hbm.at[rows])
        return _
    lax.fori_loop(0, _ZITERS, _wb, None)


_sc_gms = pl.kernel(
    _sc_body,
    mesh=plsc.VectorSubcoreMesh(core_axis_name="c", subcore_axis_name="s"),
    out_type=[jax.ShapeDtypeStruct((N, HALF), jnp.float32)] * 2,
    scratch_types=[
        pltpu.VMEM((_C,), jnp.int32),
        pltpu.VMEM((_C,), jnp.int32),
        pltpu.VMEM((_C, HALF), jnp.float32),
        pltpu.VMEM((_C, HALF), jnp.float32),
        pltpu.VMEM((_ZROWS, HALF), jnp.float32),
        pltpu.VMEM_SHARED((N, HALF), jnp.float32),
        pltpu.SemaphoreType.DMA,
    ],
    compiler_params=pltpu.CompilerParams(use_tc_tiling_on_sc=False),
)


def kernel(x, edge_index, edge_attr, params):
    src = edge_index[0]
    dst = edge_index[1]
    wp, bp = params["proj"]
    bp = bp.reshape(1, HID)

    hlo, hhi = _tc_proj(x, wp, bp)

    wargs = []
    for p in params["convs"]:
        w1, b1 = p["e1"]
        w2, b2 = p["e2"]
        wargs += [w1, b1.reshape(1, HID), w2, b2.reshape(1, HID)]
    wlos_whis = _tc_edge_mlp(edge_attr, wargs)

    for l, p in enumerate(params["convs"]):
        wlo, whi = wlos_whis[2 * l], wlos_whis[2 * l + 1]
        alo, ahi = _sc_gms(hlo, hhi, wlo, whi, src, dst)
        ws, bs = p["s"]
        wn, bn = p["n"]
        b = (bs + bn).reshape(1, HID)
        if l < 2:
            hlo, hhi = _tc_node(hlo, hhi, alo, ahi, ws, wn, b)
        else:
            msum = _tc_node_mean(hlo, hhi, alo, ahi, ws, wn, b)
    return msum[0]


# speedup vs baseline: 2.7056x; 1.6588x over previous
import functools

import jax
import jax.numpy as jnp
from jax import lax
from jax.experimental import pallas as pl
from jax.experimental.pallas import tpu as pltpu
from jax.experimental.pallas import tpu_sc as plsc

N = 50000
E = 800000
NODE_DIM = 25
HID = 64
HALF = 32


_BN = 1000
_BE = 1000


def _proj_body(x_ref, w_ref, b_ref, lo_ref, hi_ref):
    h = jnp.dot(x_ref[...], w_ref[...], preferred_element_type=jnp.float32)
    h = jnp.maximum(h + b_ref[...], 0.0)
    lo_ref[...] = h[:, :HALF]
    hi_ref[...] = h[:, HALF:]


def _edge_mlp_body(ea_ref, *refs):
    a = ea_ref[...]
    for l in range(3):
        w1, b1, w2, b2 = refs[4 * l: 4 * l + 4]
        t = jnp.maximum(
            jnp.dot(a, w1[...], preferred_element_type=jnp.float32) + b1[...], 0.0)
        y = jnp.dot(t, w2[...], preferred_element_type=jnp.float32) + b2[...]
        refs[12 + 2 * l][...] = y[:, :HALF]
        refs[12 + 2 * l + 1][...] = y[:, HALF:]


def _node_body(hlo_ref, hhi_ref, alo_ref, ahi_ref, ws_ref, wn_ref, b_ref,
               lo_ref, hi_ref):
    h = jnp.concatenate([hlo_ref[...], hhi_ref[...]], axis=1)
    a = jnp.concatenate([alo_ref[...], ahi_ref[...]], axis=1)
    y = (jnp.dot(h, ws_ref[...], preferred_element_type=jnp.float32)
         + jnp.dot(a, wn_ref[...], preferred_element_type=jnp.float32)
         + b_ref[...])
    y = jnp.maximum(y, 0.0)
    lo_ref[...] = y[:, :HALF]
    hi_ref[...] = y[:, HALF:]


def _node_mean_body(hlo_ref, hhi_ref, alo_ref, ahi_ref, ws_ref, wn_ref, b_ref,
                    out_ref):
    i = pl.program_id(0)
    h = jnp.concatenate([hlo_ref[...], hhi_ref[...]], axis=1)
    a = jnp.concatenate([alo_ref[...], ahi_ref[...]], axis=1)
    y = (jnp.dot(h, ws_ref[...], preferred_element_type=jnp.float32)
         + jnp.dot(a, wn_ref[...], preferred_element_type=jnp.float32)
         + b_ref[...])
    y = jnp.maximum(y, 0.0)

    @pl.when(i == 0)
    def _():
        out_ref[...] = jnp.zeros_like(out_ref)

    out_ref[0:1, :] += jnp.sum(y, axis=0, keepdims=True)

    @pl.when(i == pl.num_programs(0) - 1)
    def _():
        out_ref[0:1, :] *= (1.0 / N)


def _full(shape):
    return pl.BlockSpec(shape, lambda i: (0,) * len(shape))


def _tc_proj(x, wp, bp):
    g = N // _BN
    return pl.pallas_call(
        _proj_body,
        grid=(g,),
        in_specs=[
            pl.BlockSpec((_BN, NODE_DIM), lambda i: (i, 0)),
            _full((NODE_DIM, HID)),
            _full((1, HID)),
        ],
        out_specs=[pl.BlockSpec((_BN, HALF), lambda i: (i, 0))] * 2,
        out_shape=[jax.ShapeDtypeStruct((N, HALF), jnp.float32)] * 2,
    )(x, wp, bp)


def _tc_edge_mlp(ea, wargs):
    g = E // _BE
    in_specs = [pl.BlockSpec((_BE, 17), lambda i: (i, 0))]
    for _ in range(3):
        in_specs += [_full((17, HID)), _full((1, HID)),
                     _full((HID, HID)), _full((1, HID))]
    return pl.pallas_call(
        _edge_mlp_body,
        grid=(g,),
        in_specs=in_specs,
        out_specs=[pl.BlockSpec((_BE, HALF), lambda i: (i, 0))] * 6,
        out_shape=[jax.ShapeDtypeStruct((E, HALF), jnp.float32)] * 6,
    )(ea, *wargs)


def _tc_node(hlo, hhi, alo, ahi, ws, wn, b):
    g = N // _BN
    return pl.pallas_call(
        _node_body,
        grid=(g,),
        in_specs=[pl.BlockSpec((_BN, HALF), lambda i: (i, 0))] * 4
        + [_full((HID, HID)), _full((HID, HID)), _full((1, HID))],
        out_specs=[pl.BlockSpec((_BN, HALF), lambda i: (i, 0))] * 2,
        out_shape=[jax.ShapeDtypeStruct((N, HALF), jnp.float32)] * 2,
    )(hlo, hhi, alo, ahi, ws, wn, b)


def _tc_node_mean(hlo, hhi, alo, ahi, ws, wn, b):
    g = N // _BN
    return pl.pallas_call(
        _node_mean_body,
        grid=(g,),
        in_specs=[pl.BlockSpec((_BN, HALF), lambda i: (i, 0))] * 4
        + [_full((HID, HID)), _full((HID, HID)), _full((1, HID))],
        out_specs=pl.BlockSpec((8, HID), lambda i: (0, 0)),
        out_shape=jax.ShapeDtypeStruct((8, HID), jnp.float32),
    )(hlo, hhi, alo, ahi, ws, wn, b)


_C = 128
_TILES = 16
_EPT = E // _TILES
_KFULL = _EPT // _C
_TAIL = _EPT - _KFULL * _C
_UNROLL = 6
_SUPER = _KFULL // _UNROLL
_ZROWS = 40
_ZCHUNKS = N // _ZROWS
_ZITERS = -(-_ZCHUNKS // _TILES)


def _sc_body(hlo_hbm, hhi_hbm, wlo_hbm, whi_hbm, src_hbm, dst_hbm,
             olo_hbm, ohi_hbm,
             sidx, didx, tidx, hrows, wrows, zbuf, acc,
             sem_i, sem_g, sem_w, sem_s):
    cid = lax.axis_index("c")
    sid = lax.axis_index("s")
    eb = sid * _EPT

    def _for_cid(fn0, fn1):
        @pl.when(cid == 0)
        def _():
            fn0()

        @pl.when(cid == 1)
        def _():
            fn1()

    def _fire_idx(k, b2, b3):
        pltpu.async_copy(src_hbm.at[pl.ds(eb + k * _C, _C)], sidx.at[b2], sem_i)
        pltpu.async_copy(dst_hbm.at[pl.ds(eb + k * _C, _C)], didx.at[b3], sem_i)

    def _drain(sem, ref):
        pltpu.make_async_copy(hlo_hbm.at[pl.ds(0, _C)], ref, sem).wait()

    def _drain_idx():
        pltpu.make_async_copy(src_hbm.at[pl.ds(0, _C)], sidx.at[0], sem_i).wait()
        pltpu.make_async_copy(src_hbm.at[pl.ds(0, _C)], didx.at[0], sem_i).wait()

    def _fire_gw(k, b2):
        _for_cid(
            lambda: pltpu.async_copy(hlo_hbm.at[sidx.at[b2]], hrows.at[b2], sem_g),
            lambda: pltpu.async_copy(hhi_hbm.at[sidx.at[b2]], hrows.at[b2], sem_g))
        _for_cid(
            lambda: pltpu.async_copy(wlo_hbm.at[pl.ds(eb + k * _C, _C)],
                                     wrows.at[b2], sem_w),
            lambda: pltpu.async_copy(whi_hbm.at[pl.ds(eb + k * _C, _C)],
                                     wrows.at[b2], sem_w))

    def _zb(r, _):
        for c in range(HALF // 16):
            zbuf[r, pl.ds(16 * c, 16)] = jnp.zeros((16,), jnp.float32)
        return _
    lax.fori_loop(0, _ZROWS, _zb, None)

    def _zero(j, _):
        ch = j * _TILES + sid

        @pl.when(ch < _ZCHUNKS)
        def _():
            pltpu.sync_copy(zbuf, acc.at[pl.ds(ch * _ZROWS, _ZROWS)])
        return _
    lax.fori_loop(0, _ZITERS, _zero, None)

    plsc.subcore_barrier()

    _fire_idx(0, 0, 0)
    _drain_idx()
    _fire_gw(0, 0)
    _fire_idx(1, 1, 1)

    def _super(s, _):
        for j in range(_UNROLL):
            b2, nb2 = j % 2, (j + 1) % 2
            b3 = j % 3
            k = s * _UNROLL + j

            if j == 0:
                @pl.when(s > 0)
                def _():
                    _drain(sem_s, wrows.at[nb2])
            else:
                _drain(sem_s, wrows.at[nb2])

            def _s2():
                _drain_idx()
                _fire_gw(k + 1, nb2)
            if j == _UNROLL - 1:
                @pl.when(s < _SUPER - 1)
                def _():
                    _s2()
            else:
                _s2()

            _drain(sem_g, hrows.at[b2])
            _drain(sem_w, wrows.at[b2])

            def _s4():
                _fire_idx(k + 2, b2, (j + 2) % 3)
            if j >= _UNROLL - 2:
                @pl.when(s < _SUPER - 1)
                def _():
                    _s4()
            else:
                _s4()

            def _mul(r, _):
                for c in range(HALF // 16):
                    sl = pl.ds(16 * c, 16)
                    wrows[b2, r, sl] = wrows[b2, r, sl] * hrows[b2, r, sl]
                return _
            lax.fori_loop(0, _C, _mul, None)

            pltpu.async_copy(wrows.at[b2], acc.at[didx.at[b3]], sem_s, add=True)
        return _

    lax.fori_loop(0, _SUPER, _super, None)
    _drain(sem_s, wrows.at[1])

    tb = eb + _KFULL * _C
    pltpu.sync_copy(src_hbm.at[pl.ds(tb, _TAIL)], sidx.at[0, pl.ds(0, _TAIL)])
    pltpu.sync_copy(dst_hbm.at[pl.ds(tb, _TAIL)], tidx)
    _for_cid(
        lambda: pltpu.async_copy(hlo_hbm.at[sidx.at[0, pl.ds(0, _TAIL)]],
                                 hrows.at[0, pl.ds(0, _TAIL)], sem_g).wait(),
        lambda: pltpu.async_copy(hhi_hbm.at[sidx.at[0, pl.ds(0, _TAIL)]],
                                 hrows.at[0, pl.ds(0, _TAIL)], sem_g).wait())
    _for_cid(
        lambda: pltpu.sync_copy(wlo_hbm.at[pl.ds(tb, _TAIL)],
                                wrows.at[0, pl.ds(0, _TAIL)]),
        lambda: pltpu.sync_copy(whi_hbm.at[pl.ds(tb, _TAIL)],
                                wrows.at[0, pl.ds(0, _TAIL)]))

    def _mul_t(r, _):
        for c in range(HALF // 16):
            sl = pl.ds(16 * c, 16)
            wrows[0, r, sl] = wrows[0, r, sl] * hrows[0, r, sl]
        return _
    lax.fori_loop(0, _TAIL, _mul_t, None)
    pltpu.sync_copy(wrows.at[0, pl.ds(0, _TAIL)], acc.at[tidx], add=True)

    plsc.subcore_barrier()

    def _wb(j, _):
        ch = j * _TILES + sid

        @pl.when(ch < _ZCHUNKS)
        def _():
            rows = pl.ds(ch * _ZROWS, _ZROWS)

            @pl.when(cid == 0)
            def _():
                pltpu.sync_copy(acc.at[rows], olo_hbm.at[rows])

            @pl.when(cid == 1)
            def _():
                pltpu.sync_copy(acc.at[rows], ohi_hbm.at[rows])
        return _
    lax.fori_loop(0, _ZITERS, _wb, None)


_sc_gms = pl.kernel(
    _sc_body,
    mesh=plsc.VectorSubcoreMesh(core_axis_name="c", subcore_axis_name="s"),
    out_type=[jax.ShapeDtypeStruct((N, HALF), jnp.float32)] * 2,
    scratch_types=[
        pltpu.VMEM((2, _C), jnp.int32),
        pltpu.VMEM((3, _C), jnp.int32),
        pltpu.VMEM((_TAIL,), jnp.int32),
        pltpu.VMEM((2, _C, HALF), jnp.float32),
        pltpu.VMEM((2, _C, HALF), jnp.float32),
        pltpu.VMEM((_ZROWS, HALF), jnp.float32),
        pltpu.VMEM_SHARED((N, HALF), jnp.float32),
        pltpu.SemaphoreType.DMA,
        pltpu.SemaphoreType.DMA,
        pltpu.SemaphoreType.DMA,
        pltpu.SemaphoreType.DMA,
    ],
    compiler_params=pltpu.CompilerParams(use_tc_tiling_on_sc=False),
)


def kernel(x, edge_index, edge_attr, params):
    src = edge_index[0]
    dst = edge_index[1]
    wp, bp = params["proj"]
    bp = bp.reshape(1, HID)

    hlo, hhi = _tc_proj(x, wp, bp)

    wargs = []
    for p in params["convs"]:
        w1, b1 = p["e1"]
        w2, b2 = p["e2"]
        wargs += [w1, b1.reshape(1, HID), w2, b2.reshape(1, HID)]
    wlos_whis = _tc_edge_mlp(edge_attr, wargs)

    for l, p in enumerate(params["convs"]):
        wlo, whi = wlos_whis[2 * l], wlos_whis[2 * l + 1]
        alo, ahi = _sc_gms(hlo, hhi, wlo, whi, src, dst)
        ws, bs = p["s"]
        wn, bn = p["n"]
        b = (bs + bn).reshape(1, HID)
        if l < 2:
            hlo, hhi = _tc_node(hlo, hhi, alo, ahi, ws, wn, b)
        else:
            msum = _tc_node_mean(hlo, hhi, alo, ahi, ws, wn, b)
    return msum[0]
